# baseline (speedup 1.0000x reference)
import functools

import jax
import jax.numpy as jnp
from jax import lax
from jax.experimental import pallas as pl
from jax.experimental.pallas import tpu as pltpu

_TOPK = 0.1
_LAYER = 2
_T = 512
_K = 51


_RC = 128


def _dense_body(te_ref, fr_ref, tgt_ref, attn_ref, out_ref, w_ref):
    fr = fr_ref[0]
    tgt = tgt_ref[0]
    M = attn_ref.shape[-1]

    fr_norm = jnp.sqrt(jnp.sum(fr * fr))

    def _w_chunk(i, carry):
        te = te_ref[0, pl.ds(i * _RC, _RC), :]
        num = jnp.sum(te * fr, axis=1, keepdims=True)
        te_norm = jnp.sqrt(jnp.sum(te * te, axis=1, keepdims=True))
        w_ref[pl.ds(i * _RC, _RC), :] = num / jnp.maximum(te_norm * fr_norm, 1e-8)
        return carry

    lax.fori_loop(0, _T // _RC, _w_chunk, 0)

    w_col = w_ref[:, :]
    w_row = w_col.reshape(1, _T)

    seq_len = jnp.sum((tgt != 0).astype(jnp.int32))
    cc = jnp.ceil(seq_len.astype(jnp.float32) * _TOPK).astype(jnp.int32)
    cc = jnp.minimum(cc, _K)

    def _chunk(i, run_max):
        wt = w_ref[pl.ds(i * _RC, _RC), :]
        it = lax.broadcasted_iota(jnp.int32, (_RC, _T), 0) + i * _RC
        ip = lax.broadcasted_iota(jnp.int32, (_RC, _T), 1)
        beats = (w_row > wt) | ((w_row == wt) & (ip < it))
        rank = jnp.sum(beats.astype(jnp.int32), axis=1, keepdims=True)
        sel = rank < cc

        acc = attn_ref[0, 0, 0, pl.ds(i * _RC, _RC), :]
        for h in range(1, 8):
            acc = acc + attn_ref[0, 0, h, pl.ds(i * _RC, _RC), :]
        vals = jnp.maximum(wt * (acc * 0.125), 0.0)
        vals = jnp.where(sel, vals, 0.0)
        return jnp.maximum(run_max, jnp.max(vals, axis=0, keepdims=True))

    tot = lax.fori_loop(0, _T // _RC, _chunk, jnp.zeros((1, M), jnp.float32))

    shifted = tot - jnp.min(tot)
    div = jnp.clip(jnp.max(shifted), 1e-12, 1.0)
    out_ref[0, 0] = (shifted / div)[0]


def kernel(fore_map, fore_rep_encoded, target_embed, align_attns, targets):
    B, T, D = target_embed.shape
    M = align_attns.shape[-1]
    H = align_attns.shape[2]

    fr3 = fore_rep_encoded.reshape(B, 1, D)
    tgt3 = targets.reshape(B, 1, T)

    total_attn = pl.pallas_call(
        _dense_body,
        grid=(B,),
        in_specs=[
            pl.BlockSpec((1, T, D), lambda b: (b, 0, 0)),
            pl.BlockSpec((1, 1, D), lambda b: (b, 0, 0)),
            pl.BlockSpec((1, 1, T), lambda b: (b, 0, 0)),
            pl.BlockSpec((1, 1, H, T, M), lambda b: (_LAYER, b, 0, 0, 0)),
        ],
        out_specs=pl.BlockSpec((1, 1, M), lambda b: (b, 0, 0)),
        out_shape=jax.ShapeDtypeStruct((B, 1, M), jnp.float32),
        scratch_shapes=[pltpu.VMEM((T, 1), jnp.float32)],
    )(target_embed, fr3, tgt3, align_attns)

    return (jnp.squeeze(fore_map, axis=1), total_attn.reshape(B, M))


# speedup vs baseline: 6.7063x; 6.7063x over previous
import jax
import jax.numpy as jnp
from jax import lax
from jax.experimental import pallas as pl
from jax.experimental.pallas import tpu as pltpu

_TOPK = 0.1
_LAYER = 2
_K = 51
_H = 8


def _fused_body(te_ref, fr_ref, tgt_ref, attn_ref, out_ref):
    T, D = te_ref.shape[1], te_ref.shape[2]
    te = te_ref[0]
    fr = fr_ref[0]
    tgt = tgt_ref[0]

    dn = (((1,), (1,)), ((), ()))
    num_row = lax.dot_general(fr, te, dn,
                              preferred_element_type=jnp.float32)
    ones_d = jnp.ones((1, D), jnp.float32)
    sqn_row = lax.dot_general(ones_d, te * te, dn,
                              preferred_element_type=jnp.float32)
    fr_norm = jnp.sqrt(jnp.sum(fr * fr))
    w_row = num_row / jnp.maximum(jnp.sqrt(sqn_row) * fr_norm, 1e-8)
    w_col = w_row.reshape(T, 1)

    ic = lax.broadcasted_iota(jnp.int32, (T, T), 0)
    ir = lax.broadcasted_iota(jnp.int32, (T, T), 1)
    beats = (w_col > w_row) | ((w_col == w_row) & (ic < ir))
    beats_f = jnp.where(beats, 1.0, 0.0)
    ones_t = jnp.ones((1, T), jnp.float32)
    rank_row = lax.dot_general(ones_t, beats_f, (((1,), (0,)), ((), ())),
                               preferred_element_type=jnp.float32)

    seq_len = jnp.sum((tgt != 0).astype(jnp.float32))
    cc = jnp.minimum(jnp.ceil(seq_len * _TOPK), float(_K))

    wm_col = jnp.where(rank_row < cc, w_row, 0.0).reshape(T, 1)

    acc = attn_ref[0, 0, 0]
    for h in range(1, _H):
        acc = acc + attn_ref[0, 0, h]
    vals = jnp.maximum(wm_col * (acc * (1.0 / _H)), 0.0)
    tot = jnp.max(vals, axis=0, keepdims=True)

    shifted = tot - jnp.min(tot)
    div = jnp.clip(jnp.max(shifted), 1e-12, 1.0)
    out_ref[0, 0] = shifted[0] / div


def kernel(fore_map, fore_rep_encoded, target_embed, align_attns, targets):
    B, T, D = target_embed.shape
    M = align_attns.shape[-1]

    fr3 = fore_rep_encoded.reshape(B, 1, D)
    tgt3 = targets.reshape(B, 1, T)

    total_attn = pl.pallas_call(
        _fused_body,
        grid=(B,),
        in_specs=[
            pl.BlockSpec((1, T, D), lambda b: (b, 0, 0)),
            pl.BlockSpec((1, 1, D), lambda b: (b, 0, 0)),
            pl.BlockSpec((1, 1, T), lambda b: (b, 0, 0)),
            pl.BlockSpec((1, 1, _H, T, M), lambda b: (_LAYER, b, 0, 0, 0)),
        ],
        out_specs=pl.BlockSpec((1, 1, M), lambda b: (b, 0, 0)),
        out_shape=jax.ShapeDtypeStruct((B, 1, M), jnp.float32),
    )(target_embed, fr3, tgt3, align_attns)

    return (jnp.squeeze(fore_map, axis=1), total_attn.reshape(B, M))


# speedup vs baseline: 6.7316x; 1.0038x over previous
import jax
import jax.numpy as jnp
from jax import lax
from jax.experimental import pallas as pl
from jax.experimental.pallas import tpu as pltpu

_TOPK = 0.1
_LAYER = 2
_K = 51
_H = 8


def _fused_body(te_ref, fr_ref, tgt_ref, *refs):
    attn_refs = refs[:_H]
    out_ref = refs[_H]
    T, D = te_ref.shape[1], te_ref.shape[2]
    te = te_ref[0]
    fr = fr_ref[0]
    tgt = tgt_ref[0]

    dn = (((1,), (1,)), ((), ()))
    num_row = lax.dot_general(fr, te, dn,
                              preferred_element_type=jnp.float32)
    ones_d = jnp.ones((1, D), jnp.float32)
    sqn_row = lax.dot_general(ones_d, te * te, dn,
                              preferred_element_type=jnp.float32)
    fr_norm = jnp.sqrt(jnp.sum(fr * fr))
    w_row = num_row / jnp.maximum(jnp.sqrt(sqn_row) * fr_norm, 1e-8)
    w_col = w_row.reshape(T, 1)

    ic = lax.broadcasted_iota(jnp.int32, (T, T), 0)
    ir = lax.broadcasted_iota(jnp.int32, (T, T), 1)
    beats = (w_col > w_row) | ((w_col == w_row) & (ic < ir))
    beats_f = jnp.where(beats, 1.0, 0.0)
    ones_t = jnp.ones((1, T), jnp.float32)
    rank_row = lax.dot_general(ones_t, beats_f, (((1,), (0,)), ((), ())),
                               preferred_element_type=jnp.float32)

    seq_len = jnp.sum((tgt != 0).astype(jnp.float32))
    cc = jnp.minimum(jnp.ceil(seq_len * _TOPK), float(_K))

    wm_col = jnp.where(rank_row < cc, w_row, 0.0).reshape(T, 1)

    acc = attn_refs[0][0, 0, 0]
    for h in range(1, _H):
        acc = acc + attn_refs[h][0, 0, 0]
    vals = jnp.maximum(wm_col * (acc * (1.0 / _H)), 0.0)
    tot = jnp.max(vals, axis=0, keepdims=True)

    shifted = tot - jnp.min(tot)
    div = jnp.clip(jnp.max(shifted), 1e-12, 1.0)
    out_ref[0, 0] = shifted[0] / div


def _mk_attn_spec(h, T, M):
    return pl.BlockSpec((1, 1, 1, T, M), lambda b, _h=h: (_LAYER, b, _h, 0, 0))


def kernel(fore_map, fore_rep_encoded, target_embed, align_attns, targets):
    B, T, D = target_embed.shape
    M = align_attns.shape[-1]

    fr3 = fore_rep_encoded.reshape(B, 1, D)
    tgt3 = targets.reshape(B, 1, T)

    total_attn = pl.pallas_call(
        _fused_body,
        grid=(B,),
        in_specs=[
            pl.BlockSpec((1, T, D), lambda b: (b, 0, 0)),
            pl.BlockSpec((1, 1, D), lambda b: (b, 0, 0)),
            pl.BlockSpec((1, 1, T), lambda b: (b, 0, 0)),
        ] + [_mk_attn_spec(h, T, M) for h in range(_H)],
        out_specs=pl.BlockSpec((1, 1, M), lambda b: (b, 0, 0)),
        out_shape=jax.ShapeDtypeStruct((B, 1, M), jnp.float32),
    )(target_embed, fr3, tgt3, *([align_attns] * _H))

    return (jnp.squeeze(fore_map, axis=1), total_attn.reshape(B, M))
